# Initial kernel scaffold; baseline (speedup 1.0000x reference)
import jax
import jax.numpy as jnp
from jax.experimental import pallas as pl


def kernel(pos, x, batch, edge_index, params):
    raise NotImplementedError("write your pallas kernel here")


# speedup vs baseline: 1.3373x; 1.3373x over previous
import functools
import math

import jax
import jax.numpy as jnp
from jax.experimental import pallas as pl
from jax.experimental.pallas import tpu as pltpu

N_BASIS = 10
MAX_RADIUS = 2.0
H_STEP = 0.1
CAT = 96
E_BLOCK = 4000


def _silu(v):
    return v * jax.nn.sigmoid(v)


def _edge_chain_body(emb_ref, gd_ref, gs_ref,
                     a1_ref, a1b_ref, a2_ref, a2b_ref,
                     b1_ref, b1b_ref, b2_ref, b2b_ref,
                     d1g_ref, d1a_ref, d1b_ref, d2a_ref, d2ab_ref,
                     d2b_ref, d2bb_ref,
                     c1_ref, c1b_ref, c2a_ref, c2ab_ref, c2b_ref, c2bb_ref,
                     pd_ref, ps_ref):
    f32 = jnp.float32
    emb = emb_ref[...]
    gd = gd_ref[...]
    gs = gs_ref[...]
    hA = _silu(jnp.dot(emb, a1_ref[...], preferred_element_type=f32) + a1b_ref[...])
    WA = jnp.dot(hA, a2_ref[...], preferred_element_type=f32) + a2b_ref[...]
    hB = _silu(jnp.dot(emb, b1_ref[...], preferred_element_type=f32) + b1b_ref[...])
    WB = jnp.dot(hB, b2_ref[...], preferred_element_type=f32) + b2b_ref[...]
    gradX = WA * (gd - gs)
    aveX = WB * (gd + gs) * 0.5
    t = jnp.tanh(jnp.dot(gradX, d1g_ref[...], preferred_element_type=f32)
                 + jnp.dot(aveX, d1a_ref[...], preferred_element_type=f32)
                 + d1b_ref[...])
    dxe_a = jnp.dot(t, d2a_ref[...], preferred_element_type=f32) + d2ab_ref[...]
    dxe_b = jnp.dot(t, d2b_ref[...], preferred_element_type=f32) + d2bb_ref[...]
    hC = _silu(jnp.dot(emb, c1_ref[...], preferred_element_type=f32) + c1b_ref[...])
    WCa = jnp.dot(hC, c2a_ref[...], preferred_element_type=f32) + c2ab_ref[...]
    WCb = jnp.dot(hC, c2b_ref[...], preferred_element_type=f32) + c2bb_ref[...]
    a = WCa * dxe_a
    b = 0.5 * (WCb * dxe_b)
    pd_ref[...] = a + b
    ps_ref[...] = b - a


def _row(v):
    return v.reshape(1, -1)


@functools.partial(jax.jit, static_argnames=())
def _edge_chain(edge_emb, gd, gs, lp):
    E = edge_emb.shape[0]
    grid = (E // E_BLOCK,)
    eb = lambda w: pl.BlockSpec((E_BLOCK, w), lambda i: (i, 0))
    wb = lambda p: pl.BlockSpec(p.shape, lambda i: (0,) * p.ndim)
    a1, a1b = lp['filtA'][0]
    a2, a2b = lp['filtA'][1]
    b1, b1b = lp['filtB'][0]
    b2, b2b = lp['filtB'][1]
    d1, d1b = lp['dl'][0]
    d2, d2b = lp['dl'][1]
    c1, c1b = lp['filtC'][0]
    c2, c2b = lp['filtC'][1]
    d1g, d1a = d1[:CAT], d1[CAT:]
    d2a, d2b_ = d2[:, :CAT], d2[:, CAT:]
    d2ab, d2bb = d2b[:CAT], d2b[CAT:]
    c2a, c2b2 = c2[:, :CAT], c2[:, CAT:]
    c2ab, c2bb = c2b[:CAT], c2b[CAT:]
    ws = [a1, _row(a1b), a2, _row(a2b),
          b1, _row(b1b), b2, _row(b2b),
          d1g, d1a, _row(d1b), d2a, _row(d2ab), d2b_, _row(d2bb),
          c1, _row(c1b), c2a, _row(c2ab), c2b2, _row(c2bb)]
    pd, ps = pl.pallas_call(
        _edge_chain_body,
        grid=grid,
        in_specs=[eb(N_BASIS), eb(CAT), eb(CAT)] + [wb(w) for w in ws],
        out_specs=[eb(CAT), eb(CAT)],
        out_shape=[jax.ShapeDtypeStruct((E, CAT), jnp.float32),
                   jax.ShapeDtypeStruct((E, CAT), jnp.float32)],
        compiler_params=pltpu.CompilerParams(
            dimension_semantics=("arbitrary",)),
    )(edge_emb, gd, gs, *ws)
    return pd, ps


def _apply(p, v):
    return v @ p[0] + p[1]


def _double_layer(ps, v):
    return _apply(ps[1], jnp.tanh(_apply(ps[0], v)))


def _mlp(ps, v):
    return _apply(ps[1], jax.nn.silu(_apply(ps[0], v)))


def _smooth_cutoff(v):
    u = 2.0 * (v - 1.0)
    y = (1.0 - jnp.cos(jnp.pi * u)) / 2.0
    y = jnp.where(u > 0.0, 0.0, y)
    y = jnp.where(u < -1.0, 1.0, y)
    return y


def kernel(pos, x, batch, edge_index, params):
    esrc = edge_index[0]
    edst = edge_index[1]
    nnodes = pos.shape[0]
    E = esrc.shape[0]

    edge_vec = pos[esrc] - pos[edst]
    r = jnp.linalg.norm(edge_vec, axis=1)
    r_safe = jnp.clip(r, 1e-9, None)
    unit = edge_vec / r_safe[:, None]
    sh = jnp.concatenate([jnp.ones((E, 1), jnp.float32),
                          math.sqrt(3.0) * unit], axis=1)
    n = jnp.arange(1, N_BASIS + 1, dtype=jnp.float32)
    bessel = (jnp.sqrt(2.0 / MAX_RADIUS)
              * jnp.sin(n[None, :] * jnp.pi * r_safe[:, None] / MAX_RADIUS)
              / r_safe[:, None])
    edge_emb = bessel * (N_BASIS ** 0.5)
    xe = _smooth_cutoff(r / MAX_RADIUS)[:, None] * sh

    xn0 = _double_layer(params['dl_xn'], params['embed'][x])
    xe = _double_layer(params['dl_xe'], xe)
    W0 = _mlp(params['filt0'], edge_emb)
    W1 = _mlp(params['filt1'], edge_emb)
    g0 = W0 * xe
    g1 = 0.5 * (W1 * xe)
    zeros32 = jnp.zeros((nnodes, 32), jnp.float32)
    xe_eD = zeros32.at[edst].add(g0).at[esrc].add(-g0)
    xe_eA = zeros32.at[edst].add(g1).at[esrc].add(g1)
    xn = jnp.concatenate([xn0, xe_eD, xe_eA], axis=1)

    for lp in params['layers']:
        gd = xn[edst]
        gs = xn[esrc]
        pd, ps = _edge_chain(edge_emb, gd, gs, lp)
        acc = jnp.zeros((nnodes, CAT), jnp.float32)
        acc = acc.at[edst].add(pd).at[esrc].add(ps)
        xn = xn - H_STEP * acc

    out = xn @ params['si_close'][0] + params['si_close'][1]
    out = jnp.sum(out, axis=0, keepdims=True) / (nnodes ** 0.5)
    return out


# speedup vs baseline: 1.6614x; 1.2424x over previous
import functools
import math

import jax
import jax.numpy as jnp
from jax import lax
from jax.experimental import pallas as pl
from jax.experimental.pallas import tpu as pltpu
from jax.experimental.pallas import tpu_sc as plsc

N_BASIS = 10
MAX_RADIUS = 2.0
H_STEP = 0.1
CAT = 96

_NC = 2
_NS = 16
_E = 800000
_SUB = 128
_MAC = 1024
_EPAD = 819200
_NMAC = _EPAD // _MAC
_MPT_G = _NMAC // (_NC * _NS)
_MPT_S = _NMAC // _NS
_N = 50000
_NP = 51200
_RPT = _NP // _NS
_DR = 640
_EB = 2048

_CW = 16
_mesh = plsc.VectorSubcoreMesh(core_axis_name="c", subcore_axis_name="s",
                               num_cores=_NC, num_subcores=_NS)
_sc_params = pltpu.CompilerParams(use_tc_tiling_on_sc=False)


def _gather_body(w, xn_hbm, idxd_hbm, idxs_hbm, gd_hbm, gs_hbm,
                 idxd_v, idxs_v, bufs, gsem, ssem):
    cid = lax.axis_index("c")
    sid = lax.axis_index("s")
    wid = sid * _NC + cid

    def macro(k, carry):
        m = wid * _MPT_G + k
        base = m * _MAC
        pltpu.sync_copy(idxd_hbm.at[m], idxd_v)
        pltpu.sync_copy(idxs_hbm.at[m], idxs_v)
        for half, (idx_v, out_hbm) in enumerate(
                ((idxd_v, gd_hbm), (idxd_v, gd_hbm),
                 (idxs_v, gs_hbm), (idxs_v, gs_hbm))):
            g = half % 2
            gets = []
            for j in range(4):
                if half > 0:
                    puts[j].wait()
                gets.append(pltpu.async_copy(
                    xn_hbm.at[idx_v.at[g * 4 + j]], bufs.at[j], gsem))
            puts = []
            for j in range(4):
                gets[j].wait()
                puts.append(pltpu.async_copy(
                    bufs.at[j],
                    out_hbm.at[pl.ds(base + (g * 4 + j) * _SUB, _SUB)], ssem))
        for p in puts:
            p.wait()
        return carry

    lax.fori_loop(0, _MPT_G, macro, 0)


def _make_sc_gather(w):
    return functools.partial(
        pl.kernel,
        out_type=[jax.ShapeDtypeStruct((_EPAD, w), jnp.float32),
                  jax.ShapeDtypeStruct((_EPAD, w), jnp.float32)],
        mesh=_mesh,
        compiler_params=_sc_params,
        scratch_types=[
            pltpu.VMEM((8, _SUB), jnp.int32),
            pltpu.VMEM((8, _SUB), jnp.int32),
            pltpu.VMEM((4, _SUB, w), jnp.float32),
            pltpu.SemaphoreType.DMA,
            pltpu.SemaphoreType.DMA,
        ],
    )(functools.partial(_gather_body, w))


_sc_gather128 = _make_sc_gather(128)


def _scatter_body(nch, pd_hbm, ps_hbm, idxd_hbm, idxs_hbm, zero_hbm, out_hbm,
                  idxd_v, idxs_v, pbufs, zbuf, dbuf, acc, plsem, scsem):
    cid = lax.axis_index("c")
    sid = lax.axis_index("s")
    pltpu.sync_copy(zero_hbm, zbuf)
    for p in range(nch // 2):
        ch = cid * (nch // 2) + p
        for q in range(_RPT // _DR):
            pltpu.sync_copy(zbuf, acc.at[pl.ds(sid * _RPT + q * _DR, _DR)])
        plsc.subcore_barrier()
        chbase = ch * _EPAD

        def macro(k, carry):
            m = sid * _MPT_S + k
            base = m * _MAC
            pltpu.sync_copy(idxd_hbm.at[m], idxd_v)
            pltpu.sync_copy(idxs_hbm.at[m], idxs_v)
            for idx_v, p_hbm in ((idxd_v, pd_hbm), (idxs_v, ps_hbm)):
                lds = []
                for j in range(8):
                    lds.append(pltpu.async_copy(
                        p_hbm.at[pl.ds(chbase + base + j * _SUB, _SUB)],
                        pbufs.at[j], plsem))
                scs = []
                for j in range(8):
                    lds[j].wait()
                    scs.append(pltpu.async_copy(
                        pbufs.at[j], acc.at[idx_v.at[j]], scsem, add=True))
                for s in scs:
                    s.wait()
            return carry

        lax.fori_loop(0, _MPT_S, macro, 0)
        plsc.subcore_barrier()
        for q in range(_RPT // _DR):
            r0 = sid * _RPT + q * _DR
            pltpu.sync_copy(acc.at[pl.ds(r0, _DR)], dbuf)
            pltpu.sync_copy(dbuf, out_hbm.at[pl.ds(ch * _NP + r0, _DR)])
        plsc.subcore_barrier()


def _make_sc_scatter(nch):
    return functools.partial(
        pl.kernel,
        out_type=jax.ShapeDtypeStruct((nch * _NP, _CW), jnp.float32),
        mesh=_mesh,
        compiler_params=_sc_params,
        scratch_types=[
            pltpu.VMEM((8, _SUB), jnp.int32),
            pltpu.VMEM((8, _SUB), jnp.int32),
            pltpu.VMEM((8, _SUB, _CW), jnp.float32),
            pltpu.VMEM((_DR, _CW), jnp.float32),
            pltpu.VMEM((_DR, _CW), jnp.float32),
            pltpu.VMEM_SHARED((_NP, _CW), jnp.float32),
            pltpu.SemaphoreType.DMA,
            pltpu.SemaphoreType.DMA,
        ],
    )(functools.partial(_scatter_body, nch))


_sc_scatter6 = _make_sc_scatter(6)
_sc_scatter4 = _make_sc_scatter(4)


def _silu(v):
    return v * jax.nn.sigmoid(v)


def _preamble_body(d8_ref,
                   xe1_ref, xe1b_ref, xe2_ref, xe2b_ref,
                   f01_ref, f01b_ref, f02_ref, f02b_ref,
                   f11_ref, f11b_ref, f12_ref, f12b_ref,
                   emb_ref, pd4_ref, ps4_ref):
    f32 = jnp.float32
    d = d8_ref[...]
    dx = d[:, 0:1]
    dy = d[:, 1:2]
    dz = d[:, 2:3]
    r2 = dx * dx + dy * dy + dz * dz
    r = jnp.sqrt(r2)
    r_safe = jnp.maximum(r, 1e-9)
    inv = 1.0 / r_safe
    sq3 = math.sqrt(3.0)
    u = 2.0 * (r / MAX_RADIUS - 1.0)
    y = (1.0 - jnp.cos(jnp.pi * u)) / 2.0
    y = jnp.where(u > 0.0, 0.0, y)
    cutoff = jnp.where(u < -1.0, 1.0, y)
    w4 = xe1_ref[...]
    pre = cutoff * (w4[0:1, :]
                    + sq3 * inv * (dx * w4[1:2, :]
                                   + dy * w4[2:3, :]
                                   + dz * w4[3:4, :]))
    t = jnp.tanh(pre + xe1b_ref[...])
    xe = jnp.dot(t, xe2_ref[...], preferred_element_type=f32) + xe2b_ref[...]
    nvec = lax.broadcasted_iota(jnp.int32, (1, N_BASIS), 1).astype(f32) + 1.0
    bes = jnp.sin(r_safe * nvec * (jnp.pi / MAX_RADIUS)) * inv
    emb = bes * (math.sqrt(2.0 / MAX_RADIUS) * (N_BASIS ** 0.5))
    h0 = _silu(jnp.dot(emb, f01_ref[...], preferred_element_type=f32) + f01b_ref[...])
    W0 = jnp.dot(h0, f02_ref[...], preferred_element_type=f32) + f02b_ref[...]
    h1 = _silu(jnp.dot(emb, f11_ref[...], preferred_element_type=f32) + f11b_ref[...])
    W1 = jnp.dot(h1, f12_ref[...], preferred_element_type=f32) + f12b_ref[...]
    row0 = pl.program_id(0) * _EB
    rid = lax.broadcasted_iota(jnp.int32, (_EB, 1), 0) + row0
    valid = rid < _E
    g0 = jnp.where(valid, W0 * xe, 0.0)
    g1 = jnp.where(valid, 0.5 * (W1 * xe), 0.0)
    emb_ref[...] = emb
    pd4_ref[0] = g0[:, :16]
    pd4_ref[1] = g0[:, 16:]
    pd4_ref[2] = g1[:, :16]
    pd4_ref[3] = g1[:, 16:]
    ps4_ref[0] = -g0[:, :16]
    ps4_ref[1] = -g0[:, 16:]
    ps4_ref[2] = g1[:, :16]
    ps4_ref[3] = g1[:, 16:]


def _tc_preamble(d8, params):
    grid = (_EPAD // _EB,)
    eb = lambda w: pl.BlockSpec((_EB, w), lambda i: (i, 0))
    c4 = lambda w: pl.BlockSpec((4, _EB, w), lambda i: (0, i, 0))
    wb = lambda p: pl.BlockSpec(p.shape, lambda i: (0,) * p.ndim)
    xe1, xe1b = params['dl_xe'][0]
    xe2, xe2b = params['dl_xe'][1]
    f01, f01b = params['filt0'][0]
    f02, f02b = params['filt0'][1]
    f11, f11b = params['filt1'][0]
    f12, f12b = params['filt1'][1]
    ws = [xe1, xe1b.reshape(1, -1), xe2, xe2b.reshape(1, -1),
          f01, f01b.reshape(1, -1), f02, f02b.reshape(1, -1),
          f11, f11b.reshape(1, -1), f12, f12b.reshape(1, -1)]
    emb, pd4, ps4 = pl.pallas_call(
        _preamble_body,
        grid=grid,
        in_specs=[eb(8)] + [wb(w) for w in ws],
        out_specs=[eb(N_BASIS), c4(16), c4(16)],
        out_shape=[jax.ShapeDtypeStruct((_EPAD, N_BASIS), jnp.float32),
                   jax.ShapeDtypeStruct((4, _EPAD, 16), jnp.float32),
                   jax.ShapeDtypeStruct((4, _EPAD, 16), jnp.float32)],
        compiler_params=pltpu.CompilerParams(
            dimension_semantics=("arbitrary",)),
    )(d8, *ws)
    return emb, pd4, ps4


def _edge_chain_body(emb_ref, gd_ref, gs_ref,
                     a1_ref, a1b_ref, a2_ref, a2b_ref,
                     b1_ref, b1b_ref, b2_ref, b2b_ref,
                     d1g_ref, d1a_ref, d1b_ref, d2a_ref, d2ab_ref,
                     d2b_ref, d2bb_ref,
                     c1_ref, c1b_ref, c2a_ref, c2ab_ref, c2b_ref, c2bb_ref,
                     pd4_ref, ps4_ref):
    f32 = jnp.float32
    emb = emb_ref[...]
    gd = gd_ref[..., :CAT]
    gs = gs_ref[..., :CAT]
    hA = _silu(jnp.dot(emb, a1_ref[...], preferred_element_type=f32) + a1b_ref[...])
    WA = jnp.dot(hA, a2_ref[...], preferred_element_type=f32) + a2b_ref[...]
    hB = _silu(jnp.dot(emb, b1_ref[...], preferred_element_type=f32) + b1b_ref[...])
    WB = jnp.dot(hB, b2_ref[...], preferred_element_type=f32) + b2b_ref[...]
    gradX = WA * (gd - gs)
    aveX = WB * (gd + gs) * 0.5
    t = jnp.tanh(jnp.dot(gradX, d1g_ref[...], preferred_element_type=f32)
                 + jnp.dot(aveX, d1a_ref[...], preferred_element_type=f32)
                 + d1b_ref[...])
    dxe_a = jnp.dot(t, d2a_ref[...], preferred_element_type=f32) + d2ab_ref[...]
    dxe_b = jnp.dot(t, d2b_ref[...], preferred_element_type=f32) + d2bb_ref[...]
    hC = _silu(jnp.dot(emb, c1_ref[...], preferred_element_type=f32) + c1b_ref[...])
    WCa = jnp.dot(hC, c2a_ref[...], preferred_element_type=f32) + c2ab_ref[...]
    WCb = jnp.dot(hC, c2b_ref[...], preferred_element_type=f32) + c2bb_ref[...]
    row0 = pl.program_id(0) * _EB
    rid = lax.broadcasted_iota(jnp.int32, (_EB, 1), 0) + row0
    valid = rid < _E
    a = jnp.where(valid, WCa * dxe_a, 0.0)
    b = jnp.where(valid, 0.5 * (WCb * dxe_b), 0.0)
    pd = a + b
    ps = b - a
    for c in range(6):
        pd4_ref[c] = pd[:, c * _CW:(c + 1) * _CW]
        ps4_ref[c] = ps[:, c * _CW:(c + 1) * _CW]


def _edge_chain(edge_emb, gd, gs, lp):
    grid = (_EPAD // _EB,)
    eb = lambda w: pl.BlockSpec((_EB, w), lambda i: (i, 0))
    c6 = lambda w: pl.BlockSpec((6, _EB, w), lambda i: (0, i, 0))
    wb = lambda p: pl.BlockSpec(p.shape, lambda i: (0,) * p.ndim)
    a1, a1b = lp['filtA'][0]
    a2, a2b = lp['filtA'][1]
    b1, b1b = lp['filtB'][0]
    b2, b2b = lp['filtB'][1]
    d1, d1b = lp['dl'][0]
    d2, d2b = lp['dl'][1]
    c1, c1b = lp['filtC'][0]
    c2, c2b = lp['filtC'][1]
    ws = [a1, a1b.reshape(1, -1), a2, a2b.reshape(1, -1),
          b1, b1b.reshape(1, -1), b2, b2b.reshape(1, -1),
          d1[:CAT], d1[CAT:], d1b.reshape(1, -1),
          d2[:, :CAT], d2b[:CAT].reshape(1, -1),
          d2[:, CAT:], d2b[CAT:].reshape(1, -1),
          c1, c1b.reshape(1, -1),
          c2[:, :CAT], c2b[:CAT].reshape(1, -1),
          c2[:, CAT:], c2b[CAT:].reshape(1, -1)]
    pd4, ps4 = pl.pallas_call(
        _edge_chain_body,
        grid=grid,
        in_specs=[eb(N_BASIS), eb(128), eb(128)] + [wb(w) for w in ws],
        out_specs=[c6(16), c6(16)],
        out_shape=[jax.ShapeDtypeStruct((6, _EPAD, 16), jnp.float32),
                   jax.ShapeDtypeStruct((6, _EPAD, 16), jnp.float32)],
        compiler_params=pltpu.CompilerParams(
            dimension_semantics=("arbitrary",)),
    )(edge_emb, gd, gs, *ws)
    return pd4, ps4


def _apply(p, v):
    return v @ p[0] + p[1]


def _double_layer(ps, v):
    return _apply(ps[1], jnp.tanh(_apply(ps[0], v)))


def kernel(pos, x, batch, edge_index, params):
    nnodes = pos.shape[0]
    f32 = jnp.float32
    npad = _EPAD - _E
    spread = (jnp.arange(npad, dtype=jnp.int32) * 379) % nnodes
    idxs = jnp.concatenate([edge_index[0].astype(jnp.int32), spread]
                           ).reshape(_NMAC, 8, _SUB)
    idxd = jnp.concatenate([edge_index[1].astype(jnp.int32), spread]
                           ).reshape(_NMAC, 8, _SUB)

    zeros16 = jnp.zeros((_DR, _CW), f32)

    d8 = jnp.zeros((_EPAD, 8), f32).at[:_E, :3].set(pos[edge_index[0]] - pos[edge_index[1]])
    edge_emb, pd4, ps4 = _tc_preamble(d8, params)
    acc64 = _sc_scatter4(pd4.reshape(4 * _EPAD, 16), ps4.reshape(4 * _EPAD, 16),
                         idxd, idxs, zeros16)
    acc64 = acc64.reshape(4, _NP, 16)[:, :nnodes].transpose(1, 0, 2).reshape(nnodes, 64)

    xn0 = _double_layer(params['dl_xn'], params['embed'][x])
    xn = jnp.concatenate([xn0, acc64], axis=1)

    for lp in params['layers']:
        xn128 = jnp.pad(xn, ((0, 0), (0, 128 - CAT)))
        gd, gs = _sc_gather128(xn128, idxd, idxs)
        pd4, ps4 = _edge_chain(edge_emb, gd, gs, lp)
        acc = _sc_scatter6(pd4.reshape(6 * _EPAD, 16), ps4.reshape(6 * _EPAD, 16),
                           idxd, idxs, zeros16)
        acc = acc.reshape(6, _NP, 16)[:, :nnodes].transpose(1, 0, 2).reshape(nnodes, CAT)
        xn = xn - H_STEP * acc

    out = xn @ params['si_close'][0] + params['si_close'][1]
    out = jnp.sum(out, axis=0, keepdims=True) / (nnodes ** 0.5)
    return out


# speedup vs baseline: 3.4965x; 2.1045x over previous
import functools
import math

import jax
import jax.numpy as jnp
from jax import lax
from jax.experimental import pallas as pl
from jax.experimental.pallas import tpu as pltpu
from jax.experimental.pallas import tpu_sc as plsc

N_BASIS = 10
MAX_RADIUS = 2.0
H_STEP = 0.1
CAT = 96

_NC = 2
_NS = 16
_E = 800000
_SUB = 128
_MAC = 1024
_EPAD = 819200
_NMAC = _EPAD // _MAC
_MPT_G = _NMAC // (_NC * _NS)
_MPT_S = _NMAC // _NS
_N = 50000
_NP = 51200
_RPT = _NP // _NS
_DR = 640
_EB = 2048
_CW = 16

_mesh = plsc.VectorSubcoreMesh(core_axis_name="c", subcore_axis_name="s",
                               num_cores=_NC, num_subcores=_NS)
_sc_params = pltpu.CompilerParams(use_tc_tiling_on_sc=False)


def _gather_body(xn_hbm, idxd_hbm, idxs_hbm, gd_hbm, gs_hbm,
                 idxd_v, idxs_v, bufs, gsem, ssem):
    cid = lax.axis_index("c")
    sid = lax.axis_index("s")
    wid = sid * _NC + cid

    def macro(k, carry):
        m = wid * _MPT_G + k
        base = m * _MAC
        pltpu.sync_copy(idxd_hbm.at[m], idxd_v)
        pltpu.sync_copy(idxs_hbm.at[m], idxs_v)
        for half, (idx_v, out_hbm) in enumerate(
                ((idxd_v, gd_hbm), (idxd_v, gd_hbm),
                 (idxs_v, gs_hbm), (idxs_v, gs_hbm))):
            g = half % 2
            if half > 0:
                for p in puts:
                    p.wait()
            gets = []
            for j in range(4):
                gets.append(pltpu.async_copy(
                    xn_hbm.at[idx_v.at[g * 4 + j]], bufs.at[j], gsem))
            for gt in gets:
                gt.wait()
            puts = []
            for j in range(4):
                puts.append(pltpu.async_copy(
                    bufs.at[j],
                    out_hbm.at[pl.ds(base + (g * 4 + j) * _SUB, _SUB)], ssem))
        for p in puts:
            p.wait()
        return carry

    lax.fori_loop(0, _MPT_G, macro, 0)


def _make_sc_gather(w):
    return functools.partial(
        pl.kernel,
        out_type=[jax.ShapeDtypeStruct((_EPAD, w), jnp.float32),
                  jax.ShapeDtypeStruct((_EPAD, w), jnp.float32)],
        mesh=_mesh,
        compiler_params=_sc_params,
        scratch_types=[
            pltpu.VMEM((8, _SUB), jnp.int32),
            pltpu.VMEM((8, _SUB), jnp.int32),
            pltpu.VMEM((4, _SUB, w), jnp.float32),
            pltpu.SemaphoreType.DMA,
            pltpu.SemaphoreType.DMA,
        ],
    )(_gather_body)


_sc_gather128 = _make_sc_gather(128)
_sc_gather16 = _make_sc_gather(16)


def _scatter_body(nch, pd_hbm, ps_hbm, idxd_hbm, idxs_hbm, zero_hbm, out_hbm,
                  idxd_v, idxs_v, pbufs, zbuf, dbuf, acc, plsem, scsem):
    cid = lax.axis_index("c")
    sid = lax.axis_index("s")
    pltpu.sync_copy(zero_hbm, zbuf)
    for CID in range(_NC):
        @pl.when(cid == CID)
        def _core():
            for p in range(nch // 2):
                ch = CID * (nch // 2) + p
                c0 = ch * _CW
                for q in range(_RPT // _DR):
                    pltpu.sync_copy(zbuf, acc.at[pl.ds(sid * _RPT + q * _DR, _DR)])
                plsc.subcore_barrier()

                def macro(k, carry):
                    m = sid * _MPT_S + k
                    base = m * _MAC
                    pltpu.sync_copy(idxd_hbm.at[m], idxd_v)
                    pltpu.sync_copy(idxs_hbm.at[m], idxs_v)
                    for idx_v, p_hbm in ((idxd_v, pd_hbm), (idxs_v, ps_hbm)):
                        lds = []
                        for j in range(8):
                            lds.append(pltpu.async_copy(
                                p_hbm.at[pl.ds(base + j * _SUB, _SUB),
                                         pl.ds(c0, _CW)],
                                pbufs.at[j], plsem))
                        for ld in lds:
                            ld.wait()
                        scs = []
                        for j in range(8):
                            scs.append(pltpu.async_copy(
                                pbufs.at[j], acc.at[idx_v.at[j]], scsem,
                                add=True))
                        for s in scs:
                            s.wait()
                    return carry

                lax.fori_loop(0, _MPT_S, macro, 0)
                plsc.subcore_barrier()
                for q in range(_RPT // _DR):
                    r0 = sid * _RPT + q * _DR
                    pltpu.sync_copy(acc.at[pl.ds(r0, _DR)], dbuf)
                    pltpu.sync_copy(dbuf, out_hbm.at[pl.ds(ch * _NP + r0, _DR)])
                plsc.subcore_barrier()


def _make_sc_scatter(nch):
    return functools.partial(
        pl.kernel,
        out_type=jax.ShapeDtypeStruct((nch * _NP, _CW), jnp.float32),
        mesh=_mesh,
        compiler_params=_sc_params,
        scratch_types=[
            pltpu.VMEM((8, _SUB), jnp.int32),
            pltpu.VMEM((8, _SUB), jnp.int32),
            pltpu.VMEM((8, _SUB, _CW), jnp.float32),
            pltpu.VMEM((_DR, _CW), jnp.float32),
            pltpu.VMEM((_DR, _CW), jnp.float32),
            pltpu.VMEM_SHARED((_NP, _CW), jnp.float32),
            pltpu.SemaphoreType.DMA,
            pltpu.SemaphoreType.DMA,
        ],
    )(functools.partial(_scatter_body, nch))


_sc_scatter6 = _make_sc_scatter(6)
_sc_scatter4 = _make_sc_scatter(4)


def _silu(v):
    return v * jax.nn.sigmoid(v)


def _preamble_body(pg_d_ref, pg_s_ref,
                   xe1_ref, xe1b_ref, xe2_ref, xe2b_ref,
                   f01_ref, f01b_ref, f02_ref, f02b_ref,
                   f11_ref, f11b_ref, f12_ref, f12b_ref,
                   emb_ref, pd_ref, ps_ref):
    f32 = jnp.float32
    dx = pg_s_ref[:, 0:1] - pg_d_ref[:, 0:1]
    dy = pg_s_ref[:, 1:2] - pg_d_ref[:, 1:2]
    dz = pg_s_ref[:, 2:3] - pg_d_ref[:, 2:3]
    r2 = dx * dx + dy * dy + dz * dz
    r = jnp.sqrt(r2)
    r_safe = jnp.maximum(r, 1e-9)
    inv = 1.0 / r_safe
    sq3 = math.sqrt(3.0)
    u = 2.0 * (r / MAX_RADIUS - 1.0)
    y = (1.0 - jnp.cos(jnp.pi * u)) / 2.0
    y = jnp.where(u > 0.0, 0.0, y)
    cutoff = jnp.where(u < -1.0, 1.0, y)
    w4 = xe1_ref[...]
    pre = cutoff * (w4[0:1, :]
                    + sq3 * inv * (dx * w4[1:2, :]
                                   + dy * w4[2:3, :]
                                   + dz * w4[3:4, :]))
    t = jnp.tanh(pre + xe1b_ref[...])
    xe = jnp.dot(t, xe2_ref[...], preferred_element_type=f32) + xe2b_ref[...]
    nvec = lax.broadcasted_iota(jnp.int32, (1, N_BASIS), 1).astype(f32) + 1.0
    bes = jnp.sin(r_safe * nvec * (jnp.pi / MAX_RADIUS)) * inv
    emb = bes * (math.sqrt(2.0 / MAX_RADIUS) * (N_BASIS ** 0.5))
    h0 = _silu(jnp.dot(emb, f01_ref[...], preferred_element_type=f32) + f01b_ref[...])
    W0 = jnp.dot(h0, f02_ref[...], preferred_element_type=f32) + f02b_ref[...]
    h1 = _silu(jnp.dot(emb, f11_ref[...], preferred_element_type=f32) + f11b_ref[...])
    W1 = jnp.dot(h1, f12_ref[...], preferred_element_type=f32) + f12b_ref[...]
    row0 = pl.program_id(0) * _EB
    rid = lax.broadcasted_iota(jnp.int32, (_EB, 1), 0) + row0
    valid = rid < _E
    g0 = jnp.where(valid, W0 * xe, 0.0)
    g1 = jnp.where(valid, 0.5 * (W1 * xe), 0.0)
    emb_ref[...] = emb
    pd_ref[:, :32] = g0
    pd_ref[:, 32:64] = g1
    pd_ref[:, 64:] = jnp.zeros((_EB, 64), f32)
    ps_ref[:, :32] = -g0
    ps_ref[:, 32:64] = g1
    ps_ref[:, 64:] = jnp.zeros((_EB, 64), f32)


def _tc_preamble(pg_d, pg_s, params):
    grid = (_EPAD // _EB,)
    eb = lambda w: pl.BlockSpec((_EB, w), lambda i: (i, 0))
    wb = lambda p: pl.BlockSpec(p.shape, lambda i: (0,) * p.ndim)
    xe1, xe1b = params['dl_xe'][0]
    xe2, xe2b = params['dl_xe'][1]
    f01, f01b = params['filt0'][0]
    f02, f02b = params['filt0'][1]
    f11, f11b = params['filt1'][0]
    f12, f12b = params['filt1'][1]
    ws = [xe1, xe1b.reshape(1, -1), xe2, xe2b.reshape(1, -1),
          f01, f01b.reshape(1, -1), f02, f02b.reshape(1, -1),
          f11, f11b.reshape(1, -1), f12, f12b.reshape(1, -1)]
    emb, pd, ps = pl.pallas_call(
        _preamble_body,
        grid=grid,
        in_specs=[eb(16), eb(16)] + [wb(w) for w in ws],
        out_specs=[eb(N_BASIS), eb(128), eb(128)],
        out_shape=[jax.ShapeDtypeStruct((_EPAD, N_BASIS), jnp.float32),
                   jax.ShapeDtypeStruct((_EPAD, 128), jnp.float32),
                   jax.ShapeDtypeStruct((_EPAD, 128), jnp.float32)],
        compiler_params=pltpu.CompilerParams(
            dimension_semantics=("arbitrary",)),
    )(pg_d, pg_s, *ws)
    return emb, pd, ps


def _edge_chain_body(emb_ref, gd_ref, gs_ref,
                     a1_ref, a1b_ref, a2_ref, a2b_ref,
                     b1_ref, b1b_ref, b2_ref, b2b_ref,
                     d1g_ref, d1a_ref, d1b_ref, d2a_ref, d2ab_ref,
                     d2b_ref, d2bb_ref,
                     c1_ref, c1b_ref, c2a_ref, c2ab_ref, c2b_ref, c2bb_ref,
                     pd_ref, ps_ref):
    f32 = jnp.float32
    emb = emb_ref[...]
    gd = gd_ref[..., :CAT]
    gs = gs_ref[..., :CAT]
    hA = _silu(jnp.dot(emb, a1_ref[...], preferred_element_type=f32) + a1b_ref[...])
    WA = jnp.dot(hA, a2_ref[...], preferred_element_type=f32) + a2b_ref[...]
    hB = _silu(jnp.dot(emb, b1_ref[...], preferred_element_type=f32) + b1b_ref[...])
    WB = jnp.dot(hB, b2_ref[...], preferred_element_type=f32) + b2b_ref[...]
    gradX = WA * (gd - gs)
    aveX = WB * (gd + gs) * 0.5
    t = jnp.tanh(jnp.dot(gradX, d1g_ref[...], preferred_element_type=f32)
                 + jnp.dot(aveX, d1a_ref[...], preferred_element_type=f32)
                 + d1b_ref[...])
    dxe_a = jnp.dot(t, d2a_ref[...], preferred_element_type=f32) + d2ab_ref[...]
    dxe_b = jnp.dot(t, d2b_ref[...], preferred_element_type=f32) + d2bb_ref[...]
    hC = _silu(jnp.dot(emb, c1_ref[...], preferred_element_type=f32) + c1b_ref[...])
    WCa = jnp.dot(hC, c2a_ref[...], preferred_element_type=f32) + c2ab_ref[...]
    WCb = jnp.dot(hC, c2b_ref[...], preferred_element_type=f32) + c2bb_ref[...]
    row0 = pl.program_id(0) * _EB
    rid = lax.broadcasted_iota(jnp.int32, (_EB, 1), 0) + row0
    valid = rid < _E
    a = jnp.where(valid, WCa * dxe_a, 0.0)
    b = jnp.where(valid, 0.5 * (WCb * dxe_b), 0.0)
    pd_ref[:, :CAT] = a + b
    pd_ref[:, CAT:] = jnp.zeros((_EB, 128 - CAT), f32)
    ps_ref[:, :CAT] = b - a
    ps_ref[:, CAT:] = jnp.zeros((_EB, 128 - CAT), f32)


def _edge_chain(edge_emb, gd, gs, lp):
    grid = (_EPAD // _EB,)
    eb = lambda w: pl.BlockSpec((_EB, w), lambda i: (i, 0))
    wb = lambda p: pl.BlockSpec(p.shape, lambda i: (0,) * p.ndim)
    a1, a1b = lp['filtA'][0]
    a2, a2b = lp['filtA'][1]
    b1, b1b = lp['filtB'][0]
    b2, b2b = lp['filtB'][1]
    d1, d1b = lp['dl'][0]
    d2, d2b = lp['dl'][1]
    c1, c1b = lp['filtC'][0]
    c2, c2b = lp['filtC'][1]
    ws = [a1, a1b.reshape(1, -1), a2, a2b.reshape(1, -1),
          b1, b1b.reshape(1, -1), b2, b2b.reshape(1, -1),
          d1[:CAT], d1[CAT:], d1b.reshape(1, -1),
          d2[:, :CAT], d2b[:CAT].reshape(1, -1),
          d2[:, CAT:], d2b[CAT:].reshape(1, -1),
          c1, c1b.reshape(1, -1),
          c2[:, :CAT], c2b[:CAT].reshape(1, -1),
          c2[:, CAT:], c2b[CAT:].reshape(1, -1)]
    pd, ps = pl.pallas_call(
        _edge_chain_body,
        grid=grid,
        in_specs=[eb(N_BASIS), eb(128), eb(128)] + [wb(w) for w in ws],
        out_specs=[eb(128), eb(128)],
        out_shape=[jax.ShapeDtypeStruct((_EPAD, 128), jnp.float32),
                   jax.ShapeDtypeStruct((_EPAD, 128), jnp.float32)],
        compiler_params=pltpu.CompilerParams(
            dimension_semantics=("arbitrary",)),
    )(edge_emb, gd, gs, *ws)
    return pd, ps


def _apply(p, v):
    return v @ p[0] + p[1]


def _double_layer(ps, v):
    return _apply(ps[1], jnp.tanh(_apply(ps[0], v)))


def kernel(pos, x, batch, edge_index, params):
    nnodes = pos.shape[0]
    f32 = jnp.float32
    npad = _EPAD - _E
    spread = (jnp.arange(npad, dtype=jnp.int32) * 379) % nnodes
    idxs = jnp.concatenate([edge_index[0].astype(jnp.int32), spread]
                           ).reshape(_NMAC, 8, _SUB)
    idxd = jnp.concatenate([edge_index[1].astype(jnp.int32), spread]
                           ).reshape(_NMAC, 8, _SUB)

    zeros16 = jnp.zeros((_DR, _CW), f32)

    pos16 = jnp.zeros((nnodes, 16), f32).at[:, :3].set(pos)
    pg_d, pg_s = _sc_gather16(pos16, idxd, idxs)
    edge_emb, pd, ps = _tc_preamble(pg_d, pg_s, params)
    acc64 = _sc_scatter4(pd, ps, idxd, idxs, zeros16)
    acc64 = acc64.reshape(4, _NP, _CW)[:, :nnodes].transpose(1, 0, 2).reshape(nnodes, 64)

    xn0 = _double_layer(params['dl_xn'], params['embed'][x])
    xn = jnp.concatenate([xn0, acc64], axis=1)

    for lp in params['layers']:
        xn128 = jnp.pad(xn, ((0, 0), (0, 128 - CAT)))
        gd, gs = _sc_gather128(xn128, idxd, idxs)
        pd, ps = _edge_chain(edge_emb, gd, gs, lp)
        acc = _sc_scatter6(pd, ps, idxd, idxs, zeros16)
        acc = acc.reshape(6, _NP, _CW)[:, :nnodes].transpose(1, 0, 2).reshape(nnodes, CAT)
        xn = xn - H_STEP * acc

    out = xn @ params['si_close'][0] + params['si_close'][1]
    out = jnp.sum(out, axis=0, keepdims=True) / (nnodes ** 0.5)
    return out


# speedup vs baseline: 3.9792x; 1.1381x over previous
import functools
import math

import jax
import jax.numpy as jnp
from jax import lax
from jax.experimental import pallas as pl
from jax.experimental.pallas import tpu as pltpu
from jax.experimental.pallas import tpu_sc as plsc

N_BASIS = 10
MAX_RADIUS = 2.0
H_STEP = 0.1
CAT = 96

_NC = 2
_NS = 16
_E = 800000
_SUB = 128
_MAC = 1024
_EPAD = 819200
_NMAC = _EPAD // _MAC
_MPT_G = _NMAC // (_NC * _NS)
_MPT_S = _NMAC // _NS
_N = 50000
_NP = 51200
_RPT = _NP // _NS
_DR = 640
_EB = 2048
_CW = 16

_mesh = plsc.VectorSubcoreMesh(core_axis_name="c", subcore_axis_name="s",
                               num_cores=_NC, num_subcores=_NS)
_sc_params = pltpu.CompilerParams(use_tc_tiling_on_sc=False)


def _gather_body(xn_hbm, idxd_hbm, idxs_hbm, gd_hbm, gs_hbm,
                 idxd_v, idxs_v, bufs, gsem, ssem):
    cid = lax.axis_index("c")
    sid = lax.axis_index("s")
    wid = sid * _NC + cid

    def macro(k, carry):
        m = wid * _MPT_G + k
        base = m * _MAC
        pltpu.sync_copy(idxd_hbm.at[m], idxd_v)
        pltpu.sync_copy(idxs_hbm.at[m], idxs_v)
        for half, (idx_v, out_hbm) in enumerate(
                ((idxd_v, gd_hbm), (idxd_v, gd_hbm),
                 (idxs_v, gs_hbm), (idxs_v, gs_hbm))):
            g = half % 2
            if half > 0:
                for p in puts:
                    p.wait()
            gets = []
            for j in range(4):
                gets.append(pltpu.async_copy(
                    xn_hbm.at[idx_v.at[g * 4 + j]], bufs.at[j], gsem))
            for gt in gets:
                gt.wait()
            puts = []
            for j in range(4):
                puts.append(pltpu.async_copy(
                    bufs.at[j],
                    out_hbm.at[pl.ds(base + (g * 4 + j) * _SUB, _SUB)], ssem))
        for p in puts:
            p.wait()
        return carry

    lax.fori_loop(0, _MPT_G, macro, 0)


def _make_sc_gather(w):
    return functools.partial(
        pl.kernel,
        out_type=[jax.ShapeDtypeStruct((_EPAD, w), jnp.float32),
                  jax.ShapeDtypeStruct((_EPAD, w), jnp.float32)],
        mesh=_mesh,
        compiler_params=_sc_params,
        scratch_types=[
            pltpu.VMEM((8, _SUB), jnp.int32),
            pltpu.VMEM((8, _SUB), jnp.int32),
            pltpu.VMEM((4, _SUB, w), jnp.float32),
            pltpu.SemaphoreType.DMA,
            pltpu.SemaphoreType.DMA,
        ],
    )(_gather_body)


_sc_gather128 = _make_sc_gather(128)
_sc_gather16 = _make_sc_gather(16)


def _scatter_body(nch, pd_hbm, ps_hbm, idxd_hbm, idxs_hbm, zero_hbm, out_hbm,
                  idxd_v, idxs_v, pbufs, zbuf, dbuf, acc, plsem, scsem):
    cid = lax.axis_index("c")
    sid = lax.axis_index("s")
    pltpu.sync_copy(zero_hbm, zbuf)
    for CID in range(_NC):
        @pl.when(cid == CID)
        def _core():
            for p in range(nch // 2):
                ch = CID * (nch // 2) + p
                c0 = ch * _CW
                for q in range(_RPT // _DR):
                    pltpu.sync_copy(zbuf, acc.at[pl.ds(sid * _RPT + q * _DR, _DR)])
                plsc.subcore_barrier()

                def macro(k, carry):
                    m = sid * _MPT_S + k
                    base = m * _MAC
                    pltpu.sync_copy(idxd_hbm.at[m], idxd_v)
                    pltpu.sync_copy(idxs_hbm.at[m], idxs_v)
                    for idx_v, p_hbm in ((idxd_v, pd_hbm), (idxs_v, ps_hbm)):
                        lds = []
                        for j in range(8):
                            lds.append(pltpu.async_copy(
                                p_hbm.at[pl.ds(base + j * _SUB, _SUB),
                                         pl.ds(c0, _CW)],
                                pbufs.at[j], plsem))
                        for ld in lds:
                            ld.wait()
                        scs = []
                        for j in range(8):
                            scs.append(pltpu.async_copy(
                                pbufs.at[j], acc.at[idx_v.at[j]], scsem,
                                add=True))
                        for s in scs:
                            s.wait()
                    return carry

                lax.fori_loop(0, _MPT_S, macro, 0)
                plsc.subcore_barrier()
                for q in range(_RPT // _DR):
                    r0 = sid * _RPT + q * _DR
                    pltpu.sync_copy(acc.at[pl.ds(r0, _DR)], dbuf)
                    pltpu.sync_copy(dbuf, out_hbm.at[pl.ds(ch * _NP + r0, _DR)])
                plsc.subcore_barrier()


def _make_sc_scatter(nch):
    return functools.partial(
        pl.kernel,
        out_type=jax.ShapeDtypeStruct((nch * _NP, _CW), jnp.float32),
        mesh=_mesh,
        compiler_params=_sc_params,
        scratch_types=[
            pltpu.VMEM((8, _SUB), jnp.int32),
            pltpu.VMEM((8, _SUB), jnp.int32),
            pltpu.VMEM((8, _SUB, _CW), jnp.float32),
            pltpu.VMEM((_DR, _CW), jnp.float32),
            pltpu.VMEM((_DR, _CW), jnp.float32),
            pltpu.VMEM_SHARED((_NP, _CW), jnp.float32),
            pltpu.SemaphoreType.DMA,
            pltpu.SemaphoreType.DMA,
        ],
    )(functools.partial(_scatter_body, nch))


_sc_scatter6 = _make_sc_scatter(6)
_sc_scatter4 = _make_sc_scatter(4)


def _silu(v):
    return v * jax.nn.sigmoid(v)


def _preamble_body(pg_d_ref, pg_s_ref, aux_ref, emb_ref,
                   xe1_ref, xe1b_ref, xe2_ref, xe2b_ref,
                   f01_ref, f01b_ref, f02_ref, f02b_ref,
                   f11_ref, f11b_ref, f12_ref, f12b_ref,
                   pd_ref, ps_ref):
    f32 = jnp.float32
    dx = pg_s_ref[:, 0:1] - pg_d_ref[:, 0:1]
    dy = pg_s_ref[:, 1:2] - pg_d_ref[:, 1:2]
    dz = pg_s_ref[:, 2:3] - pg_d_ref[:, 2:3]
    cutoff = aux_ref[:, 0:1]
    inv = aux_ref[:, 1:2]
    emb = emb_ref[...]
    sq3 = math.sqrt(3.0)
    w4 = xe1_ref[...]
    pre = cutoff * (w4[0:1, :]
                    + sq3 * inv * (dx * w4[1:2, :]
                                   + dy * w4[2:3, :]
                                   + dz * w4[3:4, :]))
    t = jnp.tanh(pre + xe1b_ref[...])
    xe = jnp.dot(t, xe2_ref[...], preferred_element_type=f32) + xe2b_ref[...]
    h0 = _silu(jnp.dot(emb, f01_ref[...], preferred_element_type=f32) + f01b_ref[...])
    W0 = jnp.dot(h0, f02_ref[...], preferred_element_type=f32) + f02b_ref[...]
    h1 = _silu(jnp.dot(emb, f11_ref[...], preferred_element_type=f32) + f11b_ref[...])
    W1 = jnp.dot(h1, f12_ref[...], preferred_element_type=f32) + f12b_ref[...]
    row0 = pl.program_id(0) * _EB
    rid = lax.broadcasted_iota(jnp.int32, (_EB, 1), 0) + row0
    valid = rid < _E
    g0 = jnp.where(valid, W0 * xe, 0.0)
    g1 = jnp.where(valid, 0.5 * (W1 * xe), 0.0)
    pd_ref[:, :32] = g0
    pd_ref[:, 32:64] = g1
    pd_ref[:, 64:] = jnp.zeros((_EB, 64), f32)
    ps_ref[:, :32] = -g0
    ps_ref[:, 32:64] = g1
    ps_ref[:, 64:] = jnp.zeros((_EB, 64), f32)


def _tc_preamble(pg_d, pg_s, aux, emb, params):
    grid = (_EPAD // _EB,)
    eb = lambda w: pl.BlockSpec((_EB, w), lambda i: (i, 0))
    wb = lambda p: pl.BlockSpec(p.shape, lambda i: (0,) * p.ndim)
    xe1, xe1b = params['dl_xe'][0]
    xe2, xe2b = params['dl_xe'][1]
    f01, f01b = params['filt0'][0]
    f02, f02b = params['filt0'][1]
    f11, f11b = params['filt1'][0]
    f12, f12b = params['filt1'][1]
    ws = [xe1, xe1b.reshape(1, -1), xe2, xe2b.reshape(1, -1),
          f01, f01b.reshape(1, -1), f02, f02b.reshape(1, -1),
          f11, f11b.reshape(1, -1), f12, f12b.reshape(1, -1)]
    pd, ps = pl.pallas_call(
        _preamble_body,
        grid=grid,
        in_specs=[eb(16), eb(16), eb(8), eb(N_BASIS)] + [wb(w) for w in ws],
        out_specs=[eb(128), eb(128)],
        out_shape=[jax.ShapeDtypeStruct((_EPAD, 128), jnp.float32),
                   jax.ShapeDtypeStruct((_EPAD, 128), jnp.float32)],
        compiler_params=pltpu.CompilerParams(
            dimension_semantics=("arbitrary",)),
    )(pg_d, pg_s, aux, emb, *ws)
    return pd, ps


def _edge_chain_body(emb_ref, gd_ref, gs_ref,
                     a1_ref, a1b_ref, a2_ref, a2b_ref,
                     b1_ref, b1b_ref, b2_ref, b2b_ref,
                     d1g_ref, d1a_ref, d1b_ref, d2a_ref, d2ab_ref,
                     d2b_ref, d2bb_ref,
                     c1_ref, c1b_ref, c2a_ref, c2ab_ref, c2b_ref, c2bb_ref,
                     pd_ref, ps_ref):
    f32 = jnp.float32
    emb = emb_ref[...]
    gd = gd_ref[..., :CAT]
    gs = gs_ref[..., :CAT]
    hA = _silu(jnp.dot(emb, a1_ref[...], preferred_element_type=f32) + a1b_ref[...])
    WA = jnp.dot(hA, a2_ref[...], preferred_element_type=f32) + a2b_ref[...]
    hB = _silu(jnp.dot(emb, b1_ref[...], preferred_element_type=f32) + b1b_ref[...])
    WB = jnp.dot(hB, b2_ref[...], preferred_element_type=f32) + b2b_ref[...]
    gradX = WA * (gd - gs)
    aveX = WB * (gd + gs) * 0.5
    t = jnp.tanh(jnp.dot(gradX, d1g_ref[...], preferred_element_type=f32)
                 + jnp.dot(aveX, d1a_ref[...], preferred_element_type=f32)
                 + d1b_ref[...])
    dxe_a = jnp.dot(t, d2a_ref[...], preferred_element_type=f32) + d2ab_ref[...]
    dxe_b = jnp.dot(t, d2b_ref[...], preferred_element_type=f32) + d2bb_ref[...]
    hC = _silu(jnp.dot(emb, c1_ref[...], preferred_element_type=f32) + c1b_ref[...])
    WCa = jnp.dot(hC, c2a_ref[...], preferred_element_type=f32) + c2ab_ref[...]
    WCb = jnp.dot(hC, c2b_ref[...], preferred_element_type=f32) + c2bb_ref[...]
    row0 = pl.program_id(0) * _EB
    rid = lax.broadcasted_iota(jnp.int32, (_EB, 1), 0) + row0
    valid = rid < _E
    a = jnp.where(valid, WCa * dxe_a, 0.0)
    b = jnp.where(valid, 0.5 * (WCb * dxe_b), 0.0)
    pd_ref[:, :CAT] = a + b
    pd_ref[:, CAT:] = jnp.zeros((_EB, 128 - CAT), f32)
    ps_ref[:, :CAT] = b - a
    ps_ref[:, CAT:] = jnp.zeros((_EB, 128 - CAT), f32)


def _edge_chain(edge_emb, gd, gs, lp):
    grid = (_EPAD // _EB,)
    eb = lambda w: pl.BlockSpec((_EB, w), lambda i: (i, 0))
    wb = lambda p: pl.BlockSpec(p.shape, lambda i: (0,) * p.ndim)
    a1, a1b = lp['filtA'][0]
    a2, a2b = lp['filtA'][1]
    b1, b1b = lp['filtB'][0]
    b2, b2b = lp['filtB'][1]
    d1, d1b = lp['dl'][0]
    d2, d2b = lp['dl'][1]
    c1, c1b = lp['filtC'][0]
    c2, c2b = lp['filtC'][1]
    ws = [a1, a1b.reshape(1, -1), a2, a2b.reshape(1, -1),
          b1, b1b.reshape(1, -1), b2, b2b.reshape(1, -1),
          d1[:CAT], d1[CAT:], d1b.reshape(1, -1),
          d2[:, :CAT], d2b[:CAT].reshape(1, -1),
          d2[:, CAT:], d2b[CAT:].reshape(1, -1),
          c1, c1b.reshape(1, -1),
          c2[:, :CAT], c2b[:CAT].reshape(1, -1),
          c2[:, CAT:], c2b[CAT:].reshape(1, -1)]
    pd, ps = pl.pallas_call(
        _edge_chain_body,
        grid=grid,
        in_specs=[eb(N_BASIS), eb(128), eb(128)] + [wb(w) for w in ws],
        out_specs=[eb(128), eb(128)],
        out_shape=[jax.ShapeDtypeStruct((_EPAD, 128), jnp.float32),
                   jax.ShapeDtypeStruct((_EPAD, 128), jnp.float32)],
        compiler_params=pltpu.CompilerParams(
            dimension_semantics=("arbitrary",)),
    )(edge_emb, gd, gs, *ws)
    return pd, ps


def _apply(p, v):
    return v @ p[0] + p[1]


def _double_layer(ps, v):
    return _apply(ps[1], jnp.tanh(_apply(ps[0], v)))


def kernel(pos, x, batch, edge_index, params):
    nnodes = pos.shape[0]
    f32 = jnp.float32
    npad = _EPAD - _E
    spread = (jnp.arange(npad, dtype=jnp.int32) * 379) % nnodes
    idxs = jnp.concatenate([edge_index[0].astype(jnp.int32), spread]
                           ).reshape(_NMAC, 8, _SUB)
    idxd = jnp.concatenate([edge_index[1].astype(jnp.int32), spread]
                           ).reshape(_NMAC, 8, _SUB)

    zeros16 = jnp.zeros((_DR, _CW), f32)

    pos16 = jnp.zeros((nnodes, 16), f32).at[:, :3].set(pos)
    pg_d, pg_s = _sc_gather16(pos16, idxd, idxs)
    dvec = pg_s[:, :3] - pg_d[:, :3]
    r2f = jnp.sum(dvec * dvec, axis=1)
    rf = jnp.sqrt(r2f)
    rsafe = jnp.maximum(rf, 1e-9)
    invf = 1.0 / rsafe
    uf = 2.0 * (rf / MAX_RADIUS - 1.0)
    yf = (1.0 - jnp.cos(jnp.pi * uf)) / 2.0
    yf = jnp.where(uf > 0.0, 0.0, yf)
    cutf = jnp.where(uf < -1.0, 1.0, yf)
    s1 = jnp.sin(rsafe * (jnp.pi / MAX_RADIUS))
    c1 = jnp.cos(rsafe * (jnp.pi / MAX_RADIUS))
    scale = math.sqrt(2.0 / MAX_RADIUS) * (N_BASIS ** 0.5)
    two_c1 = 2.0 * c1
    harm = [s1]
    for _k in range(2, N_BASIS + 1):
        harm.append(two_c1 * harm[-1] - (harm[-2] if _k > 2 else 0.0))
    edge_emb = jnp.stack(harm, axis=1) * (invf * scale)[:, None]
    aux = jnp.stack([cutf, invf] + [jnp.zeros_like(cutf)] * 6, axis=1)
    pd, ps = _tc_preamble(pg_d, pg_s, aux, edge_emb, params)
    acc64 = _sc_scatter4(pd, ps, idxd, idxs, zeros16)
    acc64 = acc64.reshape(4, _NP, _CW)[:, :nnodes].transpose(1, 0, 2).reshape(nnodes, 64)

    xn0 = _double_layer(params['dl_xn'], params['embed'][x])
    xn = jnp.concatenate([xn0, acc64], axis=1)

    for lp in params['layers']:
        xn128 = jnp.pad(xn, ((0, 0), (0, 128 - CAT)))
        gd, gs = _sc_gather128(xn128, idxd, idxs)
        pd, ps = _edge_chain(edge_emb, gd, gs, lp)
        acc = _sc_scatter6(pd, ps, idxd, idxs, zeros16)
        acc = acc.reshape(6, _NP, _CW)[:, :nnodes].transpose(1, 0, 2).reshape(nnodes, CAT)
        xn = xn - H_STEP * acc

    out = xn @ params['si_close'][0] + params['si_close'][1]
    out = jnp.sum(out, axis=0, keepdims=True) / (nnodes ** 0.5)
    return out
